# baseline (device time: 73305 ns/iter reference)
import functools

import jax
import jax.numpy as jnp
from jax import lax
from jax.experimental import pallas as pl
from jax.experimental.pallas import tpu as pltpu

N_DEV = 4
SCALE = 0.08838834764831843
DH = 128
LOCAL_WINDOW = 128
GLOBAL_K = 32


def _fused_body(
    x_ref, wq_ref, k_ref, v_ref, wo_ref, out_ref,
    p, r1, s2, r2, s3, r3, s4, r4, acch, accq,
    send_sems, recv_sems,
):
    my = lax.axis_index("i")
    p1 = my ^ 1
    p2 = 3 - my

    sq, d = out_ref.shape
    h, q = sq // 2, sq // 4
    hq_local = k_ref.shape[0]

    qm = jnp.dot(
        x_ref[...], wq_ref[...], preferred_element_type=jnp.float32
    ).astype(jnp.bfloat16)

    QB = sq // 128

    def bias_block(qb, col0, width):
        qi = lax.broadcasted_iota(jnp.int32, (128, width), 0) + qb * 128
        ki = lax.broadcasted_iota(jnp.int32, (128, width), 1) + col0
        m = (
            (jnp.abs(qi - ki) <= LOCAL_WINDOW)
            | (ki < GLOBAL_K)
            | (qi < GLOBAL_K)
        )
        return jnp.where(m, jnp.float32(0.0), jnp.float32(-1e9))

    ctx_rows = []
    for qb in range(QB):
        if qb == 0:
            pieces = [(0, sq)]
        else:
            lo = (qb - 1) * 128
            hi = min(qb + 2, QB) * 128
            pieces = ([(0, 128)] if lo > 0 else []) + [(lo, hi - lo)]
        bias = jnp.concatenate(
            [bias_block(qb, c0, w) for c0, w in pieces], axis=1
        )
        head_parts = []
        for hh in range(hq_local):
            qh = qm[qb * 128:(qb + 1) * 128, hh * DH:(hh + 1) * DH]
            ksub = jnp.concatenate(
                [k_ref[hh, c0:c0 + w, :] for c0, w in pieces], axis=0
            )
            vsub = jnp.concatenate(
                [v_ref[hh, c0:c0 + w, :] for c0, w in pieces], axis=0
            )
            s = lax.dot_general(
                qh, ksub, (((1,), (1,)), ((), ())),
                preferred_element_type=jnp.float32,
            ) * SCALE + bias
            m = jnp.max(s, axis=1, keepdims=True)
            e = jnp.exp(s - m)
            den = jnp.sum(e, axis=1, keepdims=True)
            w = (e / den).astype(jnp.bfloat16)
            head_parts.append(
                jnp.dot(
                    w, vsub, preferred_element_type=jnp.float32
                ).astype(jnp.bfloat16)
            )
        ctx_rows.append(jnp.concatenate(head_parts, axis=1))
    ctx = jnp.concatenate(ctx_rows, axis=0)
    p[...] = jnp.dot(
        ctx, wo_ref[...], preferred_element_type=jnp.float32
    ).astype(jnp.bfloat16)

    barrier_sem = pltpu.get_barrier_semaphore()
    for nbr in [p1, p2]:
        pl.semaphore_signal(
            barrier_sem, inc=1,
            device_id=(nbr,), device_id_type=pl.DeviceIdType.MESH,
        )
    pl.semaphore_wait(barrier_sem, 2)

    cw = d // 2
    cols = [pl.ds(0, cw), pl.ds(cw, cw)]
    part = [[p1, p2, p2, p1], [p2, p1, p1, p2]]
    keep_top = [(my == 0) | (my == 3), my <= 1]
    keep_off = [jnp.where(kt, 0, h) for kt in keep_top]
    send_off = [h - ko for ko in keep_off]
    qa = [
        jnp.where(my <= 1, 0, q),
        jnp.where((my == 0) | (my == 2), 0, q),
    ]
    qb = [q - x for x in qa]

    def xfer(stage, srcs, dsts):
        rdmas = []
        for s in (0, 1):
            rdma = pltpu.make_async_remote_copy(
                src_ref=srcs[s], dst_ref=dsts[s],
                send_sem=send_sems.at[stage * 2 + s],
                recv_sem=recv_sems.at[stage * 2 + s],
                device_id=(part[s][stage],),
                device_id_type=pl.DeviceIdType.MESH,
            )
            rdma.start()
            rdmas.append(rdma)
        for rdma in rdmas:
            rdma.wait()

    xfer(
        0,
        [p.at[pl.ds(send_off[s], h), cols[s]] for s in (0, 1)],
        [r1.at[:, cols[s]] for s in (0, 1)],
    )
    for s in (0, 1):
        acch[:, cols[s]] = (
            p[pl.ds(keep_off[s], h), cols[s]].astype(jnp.float32)
            + r1[:, cols[s]].astype(jnp.float32)
        )

    for s in (0, 1):
        s2[:, cols[s]] = acch[pl.ds(qb[s], q), cols[s]].astype(jnp.bfloat16)
    xfer(
        1,
        [s2.at[:, cols[s]] for s in (0, 1)],
        [r2.at[:, cols[s]] for s in (0, 1)],
    )
    for s in (0, 1):
        accq[:, cols[s]] = (
            acch[pl.ds(qa[s], q), cols[s]]
            + r2[:, cols[s]].astype(jnp.float32)
        )

    s3[...] = accq[...].astype(jnp.bfloat16)
    xfer(
        2,
        [s3.at[:, cols[s]] for s in (0, 1)],
        [r3.at[:, cols[s]] for s in (0, 1)],
    )

    for s in (0, 1):
        s4[pl.ds(qa[s], q), cols[s]] = s3[:, cols[s]]
        s4[pl.ds(qb[s], q), cols[s]] = r3[:, cols[s]]
    xfer(
        3,
        [s4.at[:, cols[s]] for s in (0, 1)],
        [r4.at[:, cols[s]] for s in (0, 1)],
    )

    for s in (0, 1):
        out_ref[pl.ds(keep_off[s] + qa[s], q), cols[s]] = accq[:, cols[s]]
        out_ref[pl.ds(keep_off[s] + qb[s], q), cols[s]] = r3[
            :, cols[s]
        ].astype(jnp.float32)
        out_ref[pl.ds(send_off[s], h), cols[s]] = r4[:, cols[s]].astype(
            jnp.float32
        )

    @functools.partial(pl.run_scoped, sem=pltpu.SemaphoreType.REGULAR)
    def _(sem):
        for nbr in [p1, p2]:
            pl.semaphore_signal(
                sem, inc=1,
                device_id=(nbr,), device_id_type=pl.DeviceIdType.MESH,
            )
        pl.semaphore_wait(sem, 2)


def kernel(x, Wq, K_ext, V_ext, Wo):
    i = lax.axis_index("i")
    sq = x.shape[1]
    d = Wo.shape[1]
    hq_local = Wq.shape[1] // DH
    h, q = sq // 2, sq // 4
    bf = jnp.bfloat16

    xb = x[0].astype(bf)
    k = jnp.swapaxes(
        lax.dynamic_slice_in_dim(K_ext[0], i * hq_local, hq_local, axis=1),
        0, 1,
    ).astype(bf)
    v = jnp.swapaxes(
        lax.dynamic_slice_in_dim(V_ext[0], i * hq_local, hq_local, axis=1),
        0, 1,
    ).astype(bf)

    out = pl.pallas_call(
        _fused_body,
        out_shape=jax.ShapeDtypeStruct((sq, d), jnp.float32),
        in_specs=[pl.BlockSpec(memory_space=pltpu.VMEM)] * 5,
        out_specs=pl.BlockSpec(memory_space=pltpu.VMEM),
        scratch_shapes=[
            pltpu.VMEM((sq, d), bf),
            pltpu.VMEM((h, d), bf),
            pltpu.VMEM((q, d), bf),
            pltpu.VMEM((q, d), bf),
            pltpu.VMEM((q, d), bf),
            pltpu.VMEM((q, d), bf),
            pltpu.VMEM((h, d), bf),
            pltpu.VMEM((h, d), bf),
            pltpu.VMEM((h, d), jnp.float32),
            pltpu.VMEM((q, d), jnp.float32),
            pltpu.SemaphoreType.DMA((8,)),
            pltpu.SemaphoreType.DMA((8,)),
        ],
        compiler_params=pltpu.CompilerParams(collective_id=0),
    )(xb, Wq.astype(bf), k, v, Wo.astype(bf))
    return out.reshape(1, sq, d)


# device time: 62413 ns/iter; 1.1745x vs baseline; 1.1745x over previous
import functools

import jax
import jax.numpy as jnp
from jax import lax
from jax.experimental import pallas as pl
from jax.experimental.pallas import tpu as pltpu

N_DEV = 4
SCALE = 0.08838834764831843
DH = 128
LOCAL_WINDOW = 128
GLOBAL_K = 32


def _fused_body(
    x_ref, wq_ref, k_ref, v_ref, wo_ref, out_ref,
    p, r1, s2, r2, s3, r3, s4, r4, acch, accq,
    send_sems, recv_sems,
):
    my = lax.axis_index("i")
    p1 = my ^ 1
    p2 = 3 - my

    sq, d = out_ref.shape
    h, q = sq // 2, sq // 4
    hq_local = k_ref.shape[0]

    qm = jnp.dot(
        x_ref[...], wq_ref[...], preferred_element_type=jnp.float32
    ).astype(jnp.bfloat16)

    def bias_band(row0, nrows, col_pieces):
        parts = []
        for c0, w in col_pieces:
            qi = lax.broadcasted_iota(jnp.int32, (nrows, w), 0) + row0
            ki = lax.broadcasted_iota(jnp.int32, (nrows, w), 1) + c0
            m = (
                (jnp.abs(qi - ki) <= LOCAL_WINDOW)
                | (ki < GLOBAL_K)
                | (qi < GLOBAL_K)
            )
            parts.append(jnp.where(m, jnp.float32(0.0), jnp.float32(-1e9)))
        return jnp.concatenate(parts, axis=1) if len(parts) > 1 else parts[0]

    bands = [
        (0, 128, [(0, sq)]),
        (128, 384, [(0, 640)]),
        (512, 512, [(0, 128), (384, 640)]),
    ]
    biases = [bias_band(r0, nr, cp) for r0, nr, cp in bands]

    ctx_parts = []
    for hh in range(hq_local):
        band_rows = []
        for (r0, nr, cp), bias in zip(bands, biases):
            qh = qm[r0:r0 + nr, hh * DH:(hh + 1) * DH]
            if len(cp) == 1:
                c0, w = cp[0]
                ksub = k_ref[hh, c0:c0 + w, :]
                vsub = v_ref[hh, c0:c0 + w, :]
            else:
                ksub = jnp.concatenate(
                    [k_ref[hh, c0:c0 + w, :] for c0, w in cp], axis=0
                )
                vsub = jnp.concatenate(
                    [v_ref[hh, c0:c0 + w, :] for c0, w in cp], axis=0
                )
            s = lax.dot_general(
                qh, ksub, (((1,), (1,)), ((), ())),
                preferred_element_type=jnp.float32,
            ) * SCALE + bias
            m = jnp.max(s, axis=1, keepdims=True)
            e = jnp.exp(s - m)
            den = jnp.sum(e, axis=1, keepdims=True)
            w = (e / den).astype(jnp.bfloat16)
            band_rows.append(
                jnp.dot(
                    w, vsub, preferred_element_type=jnp.float32
                ).astype(jnp.bfloat16)
            )
        ctx_parts.append(jnp.concatenate(band_rows, axis=0))
    ctx = jnp.concatenate(ctx_parts, axis=1)
    p[...] = jnp.dot(
        ctx, wo_ref[...], preferred_element_type=jnp.float32
    ).astype(jnp.bfloat16)

    barrier_sem = pltpu.get_barrier_semaphore()
    for nbr in [p1, p2]:
        pl.semaphore_signal(
            barrier_sem, inc=1,
            device_id=(nbr,), device_id_type=pl.DeviceIdType.MESH,
        )
    pl.semaphore_wait(barrier_sem, 2)

    cw = d // 2
    cols = [pl.ds(0, cw), pl.ds(cw, cw)]
    part = [[p1, p2, p2, p1], [p2, p1, p1, p2]]
    keep_top = [(my == 0) | (my == 3), my <= 1]
    keep_off = [jnp.where(kt, 0, h) for kt in keep_top]
    send_off = [h - ko for ko in keep_off]
    qa = [
        jnp.where(my <= 1, 0, q),
        jnp.where((my == 0) | (my == 2), 0, q),
    ]
    qb = [q - x for x in qa]

    def xfer(stage, srcs, dsts):
        rdmas = []
        for s in (0, 1):
            rdma = pltpu.make_async_remote_copy(
                src_ref=srcs[s], dst_ref=dsts[s],
                send_sem=send_sems.at[stage * 2 + s],
                recv_sem=recv_sems.at[stage * 2 + s],
                device_id=(part[s][stage],),
                device_id_type=pl.DeviceIdType.MESH,
            )
            rdma.start()
            rdmas.append(rdma)
        for rdma in rdmas:
            rdma.wait()

    xfer(
        0,
        [p.at[pl.ds(send_off[s], h), cols[s]] for s in (0, 1)],
        [r1.at[:, cols[s]] for s in (0, 1)],
    )
    for s in (0, 1):
        acch[:, cols[s]] = (
            p[pl.ds(keep_off[s], h), cols[s]].astype(jnp.float32)
            + r1[:, cols[s]].astype(jnp.float32)
        )

    for s in (0, 1):
        s2[:, cols[s]] = acch[pl.ds(qb[s], q), cols[s]].astype(jnp.bfloat16)
    xfer(
        1,
        [s2.at[:, cols[s]] for s in (0, 1)],
        [r2.at[:, cols[s]] for s in (0, 1)],
    )
    for s in (0, 1):
        accq[:, cols[s]] = (
            acch[pl.ds(qa[s], q), cols[s]]
            + r2[:, cols[s]].astype(jnp.float32)
        )

    s3[...] = accq[...].astype(jnp.bfloat16)
    xfer(
        2,
        [s3.at[:, cols[s]] for s in (0, 1)],
        [r3.at[:, cols[s]] for s in (0, 1)],
    )

    for s in (0, 1):
        s4[pl.ds(qa[s], q), cols[s]] = s3[:, cols[s]]
        s4[pl.ds(qb[s], q), cols[s]] = r3[:, cols[s]]
    xfer(
        3,
        [s4.at[:, cols[s]] for s in (0, 1)],
        [r4.at[:, cols[s]] for s in (0, 1)],
    )

    for s in (0, 1):
        out_ref[pl.ds(keep_off[s] + qa[s], q), cols[s]] = accq[:, cols[s]]
        out_ref[pl.ds(keep_off[s] + qb[s], q), cols[s]] = r3[
            :, cols[s]
        ].astype(jnp.float32)
        out_ref[pl.ds(send_off[s], h), cols[s]] = r4[:, cols[s]].astype(
            jnp.float32
        )

    @functools.partial(pl.run_scoped, sem=pltpu.SemaphoreType.REGULAR)
    def _(sem):
        for nbr in [p1, p2]:
            pl.semaphore_signal(
                sem, inc=1,
                device_id=(nbr,), device_id_type=pl.DeviceIdType.MESH,
            )
        pl.semaphore_wait(sem, 2)


def kernel(x, Wq, K_ext, V_ext, Wo):
    i = lax.axis_index("i")
    sq = x.shape[1]
    d = Wo.shape[1]
    hq_local = Wq.shape[1] // DH
    h, q = sq // 2, sq // 4
    bf = jnp.bfloat16

    xb = x[0].astype(bf)
    k = jnp.swapaxes(
        lax.dynamic_slice_in_dim(K_ext[0], i * hq_local, hq_local, axis=1),
        0, 1,
    ).astype(bf)
    v = jnp.swapaxes(
        lax.dynamic_slice_in_dim(V_ext[0], i * hq_local, hq_local, axis=1),
        0, 1,
    ).astype(bf)

    out = pl.pallas_call(
        _fused_body,
        out_shape=jax.ShapeDtypeStruct((sq, d), jnp.float32),
        in_specs=[pl.BlockSpec(memory_space=pltpu.VMEM)] * 5,
        out_specs=pl.BlockSpec(memory_space=pltpu.VMEM),
        scratch_shapes=[
            pltpu.VMEM((sq, d), bf),
            pltpu.VMEM((h, d), bf),
            pltpu.VMEM((q, d), bf),
            pltpu.VMEM((q, d), bf),
            pltpu.VMEM((q, d), bf),
            pltpu.VMEM((q, d), bf),
            pltpu.VMEM((h, d), bf),
            pltpu.VMEM((h, d), bf),
            pltpu.VMEM((h, d), jnp.float32),
            pltpu.VMEM((q, d), jnp.float32),
            pltpu.SemaphoreType.DMA((8,)),
            pltpu.SemaphoreType.DMA((8,)),
        ],
        compiler_params=pltpu.CompilerParams(collective_id=0),
    )(xb, Wq.astype(bf), k, v, Wo.astype(bf))
    return out.reshape(1, sq, d)


# device time: 57608 ns/iter; 1.2725x vs baseline; 1.0834x over previous
import functools

import jax
import jax.numpy as jnp
from jax import lax
from jax.experimental import pallas as pl
from jax.experimental.pallas import tpu as pltpu

N_DEV = 4
SCALE = 0.08838834764831843
DH = 128
LOCAL_WINDOW = 128
GLOBAL_K = 32


def _fused_body(
    x_ref, wq_ref, k_ref, v_ref, wo_ref, out_ref,
    p_s, r1, a_s, r2, send_sems, recv_sems,
):
    my = lax.axis_index("i")
    p1 = my ^ 1
    p2 = 3 - my
    partner = [[p1, p2], [p2, p1]]

    sq, d = out_ref.shape
    hc = sq // 2
    cw = d // 2
    hq_local = k_ref.shape[0]

    barrier_sem = pltpu.get_barrier_semaphore()
    for nbr in [p1, p2]:
        pl.semaphore_signal(
            barrier_sem, inc=1,
            device_id=(nbr,), device_id_type=pl.DeviceIdType.MESH,
        )
    pl.semaphore_wait(barrier_sem, 2)

    qm = jnp.dot(
        x_ref[...], wq_ref[...], preferred_element_type=jnp.float32
    ).astype(jnp.bfloat16)

    def bias_band(row0, nrows, col_pieces):
        parts = []
        for c0, w in col_pieces:
            qi = lax.broadcasted_iota(jnp.int32, (nrows, w), 0) + row0
            ki = lax.broadcasted_iota(jnp.int32, (nrows, w), 1) + c0
            m = (
                (jnp.abs(qi - ki) <= LOCAL_WINDOW)
                | (ki < GLOBAL_K)
                | (qi < GLOBAL_K)
            )
            parts.append(jnp.where(m, jnp.float32(0.0), jnp.float32(-1e9)))
        return jnp.concatenate(parts, axis=1) if len(parts) > 1 else parts[0]

    chunk_bands = [
        [(0, 128, [(0, sq)]), (128, 384, [(0, 640)])],
        [(512, 512, [(0, 128), (384, 640)])],
    ]
    chunk_biases = [
        [bias_band(r0, nr, cp) for r0, nr, cp in bands]
        for bands in chunk_bands
    ]

    def compute_chunk(c):
        head_parts = []
        for hh in range(hq_local):
            band_rows = []
            for (r0, nr, cp), bias in zip(chunk_bands[c], chunk_biases[c]):
                qh = qm[r0:r0 + nr, hh * DH:(hh + 1) * DH]
                if len(cp) == 1:
                    c0, w = cp[0]
                    ksub = k_ref[hh, c0:c0 + w, :]
                    vsub = v_ref[hh, c0:c0 + w, :]
                else:
                    ksub = jnp.concatenate(
                        [k_ref[hh, c0:c0 + w, :] for c0, w in cp], axis=0
                    )
                    vsub = jnp.concatenate(
                        [v_ref[hh, c0:c0 + w, :] for c0, w in cp], axis=0
                    )
                s = lax.dot_general(
                    qh, ksub, (((1,), (1,)), ((), ())),
                    preferred_element_type=jnp.float32,
                ) * SCALE + bias
                m = jnp.max(s, axis=1, keepdims=True)
                e = jnp.exp(s - m)
                den = jnp.sum(e, axis=1, keepdims=True)
                w = (e / den).astype(jnp.bfloat16)
                band_rows.append(
                    jnp.dot(
                        w, vsub, preferred_element_type=jnp.float32
                    ).astype(jnp.bfloat16)
                )
            head_parts.append(
                band_rows[0] if len(band_rows) == 1
                else jnp.concatenate(band_rows, axis=0)
            )
        ctx = jnp.concatenate(head_parts, axis=1)
        p_s[c * hc:(c + 1) * hc, :] = jnp.dot(
            ctx, wo_ref[...], preferred_element_type=jnp.float32
        ).astype(jnp.bfloat16)

    def issue(stage, c, src_ref, dst_ref):
        ops = []
        for s in (0, 1):
            rows = pl.ds(c * hc, hc)
            cols = pl.ds(s * cw, cw)
            rdma = pltpu.make_async_remote_copy(
                src_ref=src_ref.at[rows, cols],
                dst_ref=dst_ref.at[rows, cols],
                send_sem=send_sems.at[stage * 4 + c * 2 + s],
                recv_sem=recv_sems.at[stage * 4 + c * 2 + s],
                device_id=(partner[s][stage],),
                device_id_type=pl.DeviceIdType.MESH,
            )
            rdma.start()
            ops.append(rdma)
        return ops

    rows_c = [pl.ds(0, hc), pl.ds(hc, hc)]

    compute_chunk(0)
    s1_c0 = issue(0, 0, p_s, r1)
    compute_chunk(1)
    s1_c1 = issue(0, 1, p_s, r1)

    for op in s1_c0:
        op.wait()
    a_s[rows_c[0], :] = (
        p_s[rows_c[0], :].astype(jnp.float32)
        + r1[rows_c[0], :].astype(jnp.float32)
    ).astype(jnp.bfloat16)
    s2_c0 = issue(1, 0, a_s, r2)

    for op in s1_c1:
        op.wait()
    a_s[rows_c[1], :] = (
        p_s[rows_c[1], :].astype(jnp.float32)
        + r1[rows_c[1], :].astype(jnp.float32)
    ).astype(jnp.bfloat16)
    s2_c1 = issue(1, 1, a_s, r2)

    for op in s2_c0:
        op.wait()
    out_ref[rows_c[0], :] = (
        a_s[rows_c[0], :].astype(jnp.float32)
        + r2[rows_c[0], :].astype(jnp.float32)
    )
    for op in s2_c1:
        op.wait()
    out_ref[rows_c[1], :] = (
        a_s[rows_c[1], :].astype(jnp.float32)
        + r2[rows_c[1], :].astype(jnp.float32)
    )

    @functools.partial(pl.run_scoped, sem=pltpu.SemaphoreType.REGULAR)
    def _(sem):
        for nbr in [p1, p2]:
            pl.semaphore_signal(
                sem, inc=1,
                device_id=(nbr,), device_id_type=pl.DeviceIdType.MESH,
            )
        pl.semaphore_wait(sem, 2)


def kernel(x, Wq, K_ext, V_ext, Wo):
    i = lax.axis_index("i")
    sq = x.shape[1]
    d = Wo.shape[1]
    hq_local = Wq.shape[1] // DH
    bf = jnp.bfloat16

    xb = x[0].astype(bf)
    k = jnp.swapaxes(
        lax.dynamic_slice_in_dim(K_ext[0], i * hq_local, hq_local, axis=1),
        0, 1,
    ).astype(bf)
    v = jnp.swapaxes(
        lax.dynamic_slice_in_dim(V_ext[0], i * hq_local, hq_local, axis=1),
        0, 1,
    ).astype(bf)

    out = pl.pallas_call(
        _fused_body,
        out_shape=jax.ShapeDtypeStruct((sq, d), jnp.float32),
        in_specs=[pl.BlockSpec(memory_space=pltpu.VMEM)] * 5,
        out_specs=pl.BlockSpec(memory_space=pltpu.VMEM),
        scratch_shapes=[
            pltpu.VMEM((sq, d), bf),
            pltpu.VMEM((sq, d), bf),
            pltpu.VMEM((sq, d), bf),
            pltpu.VMEM((sq, d), bf),
            pltpu.SemaphoreType.DMA((8,)),
            pltpu.SemaphoreType.DMA((8,)),
        ],
        compiler_params=pltpu.CompilerParams(collective_id=0),
    )(xb, Wq.astype(bf), k, v, Wo.astype(bf))
    return out.reshape(1, sq, d)
